# Initial kernel scaffold; baseline (speedup 1.0000x reference)
import jax
import jax.numpy as jnp
from jax.experimental import pallas as pl


def kernel(z, codebook):
    raise NotImplementedError("write your pallas kernel here")


# speedup vs baseline: 1.5415x; 1.5415x over previous
import functools

import jax
import jax.numpy as jnp
from jax.experimental import pallas as pl

K = 1024
DIM = 64
ROWS = 64 * 576
BLK = 1024
NBLK = ROWS // BLK


def _vq_block(z_ref, cb_ref, zq_ref, idx_ref, cnt_ref, loss_ref):
    i = pl.program_id(0)
    zb = z_ref[...]
    cb = cb_ref[...]
    logits = jax.lax.dot_general(
        zb, cb, (((1,), (1,)), ((), ())), preferred_element_type=jnp.float32)
    z2 = jnp.sum(zb * zb, axis=1, keepdims=True)
    c2 = jnp.sum(cb * cb, axis=1)
    dist = z2 + c2[None, :] - 2.0 * logits
    rowmin = jnp.min(dist, axis=1, keepdims=True)
    iota = jax.lax.broadcasted_iota(jnp.int32, dist.shape, 1)
    idx = jnp.min(jnp.where(dist == rowmin, iota, K), axis=1)
    idx_ref[0, 0, :] = idx
    onehot = (iota == idx[:, None]).astype(jnp.float32)
    zq = jax.lax.dot_general(
        onehot, cb, (((1,), (0,)), ((), ())), preferred_element_type=jnp.float32)
    diff = zq - zb
    zq_ref[...] = zb + diff
    cnt_part = jnp.sum(onehot, axis=0).reshape(8, 128)
    loss_part = jnp.sum(diff * diff).reshape(1, 1)

    @pl.when(i == 0)
    def _init():
        cnt_ref[...] = cnt_part
        loss_ref[...] = loss_part

    @pl.when(i > 0)
    def _acc():
        cnt_ref[...] += cnt_part
        loss_ref[...] += loss_part


@functools.partial(jax.jit, static_argnames=())
def kernel(z, codebook):
    z_flat = z.reshape(ROWS, DIM)
    zq, idx3, counts, loss = pl.pallas_call(
        _vq_block,
        grid=(NBLK,),
        in_specs=[
            pl.BlockSpec((BLK, DIM), lambda i: (i, 0)),
            pl.BlockSpec((K, DIM), lambda i: (0, 0)),
        ],
        out_specs=[
            pl.BlockSpec((BLK, DIM), lambda i: (i, 0)),
            pl.BlockSpec((1, 1, BLK), lambda i: (i, 0, 0)),
            pl.BlockSpec((8, 128), lambda i: (0, 0)),
            pl.BlockSpec((1, 1), lambda i: (0, 0)),
        ],
        out_shape=[
            jax.ShapeDtypeStruct((ROWS, DIM), jnp.float32),
            jax.ShapeDtypeStruct((NBLK, 1, BLK), jnp.int32),
            jax.ShapeDtypeStruct((8, 128), jnp.float32),
            jax.ShapeDtypeStruct((1, 1), jnp.float32),
        ],
    )(z_flat, codebook)
    z_q_st = zq.reshape(z.shape)
    indices = idx3.reshape(z.shape[:-1])
    n = jnp.float32(ROWS * DIM)
    loss_scalar = loss[0, 0] / n
    usage = counts.reshape(K) / jnp.float32(ROWS)
    return (z_q_st, indices, loss_scalar, loss_scalar, usage)
